# baseline (speedup 1.0000x reference)
import functools

import jax
import jax.numpy as jnp
import numpy as np
from jax import lax
from jax.experimental import pallas as pl
from jax.experimental.pallas import tpu as pltpu
from jax.experimental.pallas import tpu_sc as plsc

ROWS = 1024
COLS = 32768
TOPK = 64
LANES = 16
NWORK = 32
ROWS_PER_W = ROWS // NWORK
NVREG = COLS // LANES
CHAINS = 4
STEPS_A = NVREG // CHAINS

_I32_MIN = np.int32(-2147483648)
_I32_LOW = np.int32(0x7FFFFFFF)


def _sort_key(v):
    b = plsc.bitcast(v, jnp.int32)
    return b ^ ((b >> 31) & _I32_LOW)


def _key_to_f32(k):
    b = k ^ ((k >> 31) & _I32_LOW)
    return plsc.bitcast(b, jnp.float32)


def _ce4(ts4, u):
    t0, t1, t2, t3 = ts4
    m0 = jnp.maximum(t0, u)
    u = jnp.minimum(t0, u)
    m1 = jnp.maximum(t1, u)
    u = jnp.minimum(t1, u)
    m2 = jnp.maximum(t2, u)
    u = jnp.minimum(t2, u)
    m3 = jnp.maximum(t3, u)
    return [m0, m1, m2, m3]


def _topk_body(x_hbm, out_hbm, idx_hbm,
               row_ref, ckey_ref, cidx_ref, wkey_ref, widx_ref, oidx_ref):
    wid = lax.axis_index("s") * 2 + lax.axis_index("c")
    iota = lax.iota(jnp.int32, LANES)
    zeros_i = jnp.zeros((LANES,), jnp.int32)
    zeros_f = jnp.zeros((LANES,), jnp.float32)
    lane0 = iota == 0

    def per_row(r, _carry):
        row = wid * ROWS_PER_W + r
        pltpu.sync_copy(x_hbm.at[row], row_ref)

        def body_a(i, ts):
            base = i * (8 * LANES)
            return tuple(
                jnp.maximum(ts[c], row_ref[pl.ds(base + c * LANES, LANES)])
                for c in range(8))

        init = tuple(jnp.full((LANES,), -jnp.inf, jnp.float32)
                     for _ in range(8))
        ts = lax.fori_loop(0, NVREG // 8, body_a, init)
        t_f = jnp.min(jnp.minimum(
            jnp.minimum(jnp.minimum(ts[0], ts[1]), jnp.minimum(ts[2], ts[3])),
            jnp.minimum(jnp.minimum(ts[4], ts[5]), jnp.minimum(ts[6], ts[7]))))

        def body_b(i, off_s):
            base = i * (4 * LANES)
            base_vec = zeros_i + i * (4 * LANES)
            for c in range(4):
                v = row_ref[pl.ds(base + c * LANES, LANES)]
                msk = v >= t_f
                row_ref[pl.ds(base + c * LANES, LANES)] = zeros_f
                plsc.store_compressed(ckey_ref.at[pl.ds(off_s, LANES)],
                                      plsc.bitcast(v, jnp.int32), mask=msk)
                plsc.store_compressed(cidx_ref.at[pl.ds(off_s, LANES)],
                                      base_vec + (iota + c * LANES), mask=msk)
                off_s = off_s + plsc.all_reduce_population_count(msk)[0]
            return off_s

        ncand = lax.fori_loop(0, NVREG // 4, body_b, np.int32(0))
        nv = (ncand + LANES - 1) // LANES

        def tbody(j, _c):
            s = pl.ds(j * LANES, LANES)
            b = ckey_ref[s]
            ckey_ref[s] = b ^ ((b >> 31) & _I32_LOW)
            return _c

        lax.fori_loop(0, nv, tbody, np.int32(0))

        def count_pred(strict):
            def count(t):
                def cbody(j, acc):
                    u = ckey_ref[pl.ds(j * LANES, LANES)]
                    valid = (iota + j * LANES) < ncand
                    m = valid & ((u > t) if strict else (u >= t))
                    return acc + plsc.all_reduce_population_count(m)
                return jnp.max(lax.fori_loop(0, nv, cbody, zeros_i))
            return count

        count_ge = count_pred(False)
        count_gt = count_pred(True)

        def bbody(b, tu):
            cand_u = tu | (np.int32(1) << (np.int32(31) - b.astype(jnp.int32)))
            c = count_ge(cand_u ^ _I32_MIN)
            return jnp.where(c >= TOPK, cand_u, tu)

        t_u = lax.fori_loop(0, 32, bbody, np.int32(0))
        thr = t_u ^ _I32_MIN
        tie_take = TOPK - count_gt(thr)

        def ebody(j, carry):
            eqoff, woff = carry
            u = ckey_ref[pl.ds(j * LANES, LANES)]
            iv = cidx_ref[pl.ds(j * LANES, LANES)]
            valid = (iota + j * LANES) < ncand
            gt = valid & (u > thr)
            eq = valid & (u == thr)
            eqrank = eqoff + plsc.cumsum(eq.astype(jnp.int32)) - 1
            sel = gt | (eq & (eqrank < tie_take))
            wpos = woff + plsc.cumsum(sel.astype(jnp.int32)) - 1
            plsc.store_scatter(wkey_ref, [wpos], u, mask=sel)
            plsc.store_scatter(widx_ref, [wpos], iv, mask=sel)
            return (eqoff + plsc.all_reduce_population_count(eq),
                    woff + plsc.all_reduce_population_count(sel))

        lax.fori_loop(0, nv, ebody, (zeros_i, zeros_i))

        kvs = [wkey_ref[pl.ds(w * LANES, LANES)] for w in range(4)]
        dvs = [widx_ref[pl.ds(w * LANES, LANES)] for w in range(4)]

        for w in range(4):
            val = jnp.maximum(_key_to_f32(kvs[w]), 0.0)
            plsc.store_scatter(row_ref, [dvs[w]], val)

        for i in range(TOPK):
            ks = kvs[i // LANES][i % LANES]
            dsc = dvs[i // LANES][i % LANES]
            cnt = zeros_i
            for w in range(4):
                gm = (kvs[w] > ks) | ((kvs[w] == ks) & (dvs[w] < dsc))
                cnt = cnt + plsc.all_reduce_population_count(gm)
            rank = jnp.max(cnt)
            plsc.store_scatter(oidx_ref, [zeros_i + rank], zeros_i + dsc,
                               mask=lane0)

        pltpu.sync_copy(row_ref, out_hbm.at[row])
        pltpu.sync_copy(oidx_ref, idx_hbm.at[row])
        return _carry

    lax.fori_loop(0, ROWS_PER_W, per_row, np.int32(0))


@functools.partial(jax.jit, donate_argnums=())
def _run(x):
    mesh = plsc.VectorSubcoreMesh(core_axis_name="c", subcore_axis_name="s")
    f = pl.kernel(
        _topk_body,
        out_type=[
            jax.ShapeDtypeStruct((ROWS, COLS), jnp.float32),
            jax.ShapeDtypeStruct((ROWS, TOPK), jnp.int32),
        ],
        mesh=mesh,
        compiler_params=pltpu.CompilerParams(needs_layout_passes=False),
        scratch_types=[
            pltpu.VMEM((COLS,), jnp.float32),
            pltpu.VMEM((COLS + LANES,), jnp.int32),
            pltpu.VMEM((COLS + LANES,), jnp.int32),
            pltpu.VMEM((TOPK,), jnp.int32),
            pltpu.VMEM((TOPK,), jnp.int32),
            pltpu.VMEM((TOPK,), jnp.int32),
        ],
    )
    res, idx = f(x)
    return res, idx


def kernel(x):
    return _run(x)


# speedup vs baseline: 1.1772x; 1.1772x over previous
import functools

import jax
import jax.numpy as jnp
import numpy as np
from jax import lax
from jax.experimental import pallas as pl
from jax.experimental.pallas import tpu as pltpu
from jax.experimental.pallas import tpu_sc as plsc

ROWS = 1024
COLS = 32768
TOPK = 64
LANES = 16
NWORK = 32
ROWS_PER_W = ROWS // NWORK
NVREG = COLS // LANES
CHAINS = 4
STEPS_A = NVREG // CHAINS

_I32_MIN = np.int32(-2147483648)
_I32_LOW = np.int32(0x7FFFFFFF)


def _sort_key(v):
    b = plsc.bitcast(v, jnp.int32)
    return b ^ ((b >> 31) & _I32_LOW)


def _key_to_f32(k):
    b = k ^ ((k >> 31) & _I32_LOW)
    return plsc.bitcast(b, jnp.float32)


def _ce4(ts4, u):
    t0, t1, t2, t3 = ts4
    m0 = jnp.maximum(t0, u)
    u = jnp.minimum(t0, u)
    m1 = jnp.maximum(t1, u)
    u = jnp.minimum(t1, u)
    m2 = jnp.maximum(t2, u)
    u = jnp.minimum(t2, u)
    m3 = jnp.maximum(t3, u)
    return [m0, m1, m2, m3]


def _topk_body(x_hbm, out_hbm, idx_hbm,
               row_ref, ckey_ref, cidx_ref, wkey_ref, widx_ref, oidx_ref):
    wid = lax.axis_index("s") * 2 + lax.axis_index("c")
    iota = lax.iota(jnp.int32, LANES)
    zeros_i = jnp.zeros((LANES,), jnp.int32)
    zeros_f = jnp.zeros((LANES,), jnp.float32)
    lane0 = iota == 0

    def per_row(r, _carry):
        row = wid * ROWS_PER_W + r
        pltpu.sync_copy(x_hbm.at[row], row_ref)

        def body_a(i, ts):
            base = i * (4 * LANES)
            return tuple(
                jnp.maximum(ts[c], row_ref[pl.ds(base + c * LANES, LANES)])
                for c in range(4))

        init = tuple(jnp.full((LANES,), -jnp.inf, jnp.float32)
                     for _ in range(4))
        ts = lax.fori_loop(0, NVREG // 4, body_a, init)
        t_f = jnp.min(jnp.minimum(jnp.minimum(ts[0], ts[1]),
                                  jnp.minimum(ts[2], ts[3])))

        def body_b(i, off_s):
            base = i * (4 * LANES)
            base_vec = zeros_i + i * (4 * LANES)
            for c in range(4):
                v = row_ref[pl.ds(base + c * LANES, LANES)]
                msk = v >= t_f
                row_ref[pl.ds(base + c * LANES, LANES)] = zeros_f
                u = _sort_key(v)
                plsc.store_compressed(ckey_ref.at[pl.ds(off_s, LANES)], u,
                                      mask=msk)
                plsc.store_compressed(cidx_ref.at[pl.ds(off_s, LANES)],
                                      base_vec + (iota + c * LANES), mask=msk)
                off_s = off_s + plsc.all_reduce_population_count(msk)[0]
            return off_s

        ncand = lax.fori_loop(0, NVREG // 4, body_b, np.int32(0))
        nv = (ncand + LANES - 1) // LANES

        def count_pred(strict):
            def count(t):
                def cbody(j, acc):
                    u = ckey_ref[pl.ds(j * LANES, LANES)]
                    valid = (iota + j * LANES) < ncand
                    m = valid & ((u > t) if strict else (u >= t))
                    return acc + plsc.all_reduce_population_count(m)
                return jnp.max(lax.fori_loop(0, nv, cbody, zeros_i))
            return count

        count_ge = count_pred(False)
        count_gt = count_pred(True)

        def bbody(b, tu):
            cand_u = tu | (np.int32(1) << (np.int32(31) - b.astype(jnp.int32)))
            c = count_ge(cand_u ^ _I32_MIN)
            return jnp.where(c >= TOPK, cand_u, tu)

        t_u = lax.fori_loop(0, 32, bbody, np.int32(0))
        thr = t_u ^ _I32_MIN
        tie_take = TOPK - count_gt(thr)

        def ebody(j, carry):
            eqoff, woff = carry
            u = ckey_ref[pl.ds(j * LANES, LANES)]
            iv = cidx_ref[pl.ds(j * LANES, LANES)]
            valid = (iota + j * LANES) < ncand
            gt = valid & (u > thr)
            eq = valid & (u == thr)
            eqrank = eqoff + plsc.cumsum(eq.astype(jnp.int32)) - 1
            sel = gt | (eq & (eqrank < tie_take))
            wpos = woff + plsc.cumsum(sel.astype(jnp.int32)) - 1
            plsc.store_scatter(wkey_ref, [wpos], u, mask=sel)
            plsc.store_scatter(widx_ref, [wpos], iv, mask=sel)
            return (eqoff + plsc.all_reduce_population_count(eq),
                    woff + plsc.all_reduce_population_count(sel))

        lax.fori_loop(0, nv, ebody, (zeros_i, zeros_i))

        kvs = [wkey_ref[pl.ds(w * LANES, LANES)] for w in range(4)]
        dvs = [widx_ref[pl.ds(w * LANES, LANES)] for w in range(4)]

        for w in range(4):
            val = jnp.maximum(_key_to_f32(kvs[w]), 0.0)
            plsc.store_scatter(row_ref, [dvs[w]], val)

        for i in range(TOPK):
            ks = kvs[i // LANES][i % LANES]
            dsc = dvs[i // LANES][i % LANES]
            cnt = zeros_i
            for w in range(4):
                gm = (kvs[w] > ks) | ((kvs[w] == ks) & (dvs[w] < dsc))
                cnt = cnt + plsc.all_reduce_population_count(gm)
            rank = jnp.max(cnt)
            plsc.store_scatter(oidx_ref, [zeros_i + rank], zeros_i + dsc,
                               mask=lane0)

        pltpu.sync_copy(row_ref, out_hbm.at[row])
        pltpu.sync_copy(oidx_ref, idx_hbm.at[row])
        return _carry

    lax.fori_loop(0, ROWS_PER_W, per_row, np.int32(0))


@functools.partial(jax.jit, donate_argnums=())
def _run(x):
    mesh = plsc.VectorSubcoreMesh(core_axis_name="c", subcore_axis_name="s")
    f = pl.kernel(
        _topk_body,
        out_type=[
            jax.ShapeDtypeStruct((ROWS, COLS), jnp.float32),
            jax.ShapeDtypeStruct((ROWS, TOPK), jnp.int32),
        ],
        mesh=mesh,
        compiler_params=pltpu.CompilerParams(needs_layout_passes=False),
        scratch_types=[
            pltpu.VMEM((COLS,), jnp.float32),
            pltpu.VMEM((COLS + LANES,), jnp.int32),
            pltpu.VMEM((COLS + LANES,), jnp.int32),
            pltpu.VMEM((TOPK,), jnp.int32),
            pltpu.VMEM((TOPK,), jnp.int32),
            pltpu.VMEM((TOPK,), jnp.int32),
        ],
    )
    res, idx = f(x)
    return res, idx


def kernel(x):
    return _run(x)


# speedup vs baseline: 1.1941x; 1.0144x over previous
import functools

import jax
import jax.numpy as jnp
import numpy as np
from jax import lax
from jax.experimental import pallas as pl
from jax.experimental.pallas import tpu as pltpu
from jax.experimental.pallas import tpu_sc as plsc

ROWS = 1024
COLS = 32768
TOPK = 64
LANES = 16
NWORK = 32
ROWS_PER_W = ROWS // NWORK
NVREG = COLS // LANES
CHAINS = 4
STEPS_A = NVREG // CHAINS

_I32_MIN = np.int32(-2147483648)
_I32_LOW = np.int32(0x7FFFFFFF)


def _sort_key(v):
    b = plsc.bitcast(v, jnp.int32)
    return b ^ ((b >> 31) & _I32_LOW)


def _key_to_f32(k):
    b = k ^ ((k >> 31) & _I32_LOW)
    return plsc.bitcast(b, jnp.float32)


def _ce4(ts4, u):
    t0, t1, t2, t3 = ts4
    m0 = jnp.maximum(t0, u)
    u = jnp.minimum(t0, u)
    m1 = jnp.maximum(t1, u)
    u = jnp.minimum(t1, u)
    m2 = jnp.maximum(t2, u)
    u = jnp.minimum(t2, u)
    m3 = jnp.maximum(t3, u)
    return [m0, m1, m2, m3]


def _topk_body(x_hbm, out_hbm, idx_hbm,
               row_ref, ckey_ref, cidx_ref, wkey_ref, widx_ref, oidx_ref):
    wid = lax.axis_index("s") * 2 + lax.axis_index("c")
    iota = lax.iota(jnp.int32, LANES)
    zeros_i = jnp.zeros((LANES,), jnp.int32)
    zeros_f = jnp.zeros((LANES,), jnp.float32)
    lane0 = iota == 0

    def per_row(r, _carry):
        row = wid * ROWS_PER_W + r
        pltpu.sync_copy(x_hbm.at[row], row_ref)

        def body_a(i, ts):
            base = i * (4 * LANES)
            return tuple(
                jnp.maximum(ts[c], row_ref[pl.ds(base + c * LANES, LANES)])
                for c in range(4))

        init = tuple(jnp.full((LANES,), -jnp.inf, jnp.float32)
                     for _ in range(4))
        ts = lax.fori_loop(0, NVREG // 4, body_a, init)
        t_f = jnp.min(jnp.minimum(jnp.minimum(ts[0], ts[1]),
                                  jnp.minimum(ts[2], ts[3])))
        lo_key = jnp.min(_sort_key(jnp.minimum(jnp.minimum(ts[0], ts[1]),
                                               jnp.minimum(ts[2], ts[3]))))
        hi_key = jnp.max(_sort_key(jnp.maximum(jnp.maximum(ts[0], ts[1]),
                                               jnp.maximum(ts[2], ts[3]))))

        def body_b(i, off_s):
            base = i * (4 * LANES)
            base_vec = zeros_i + i * (4 * LANES)
            for c in range(4):
                v = row_ref[pl.ds(base + c * LANES, LANES)]
                msk = v >= t_f
                row_ref[pl.ds(base + c * LANES, LANES)] = zeros_f
                u = _sort_key(v)
                plsc.store_compressed(ckey_ref.at[pl.ds(off_s, LANES)], u,
                                      mask=msk)
                plsc.store_compressed(cidx_ref.at[pl.ds(off_s, LANES)],
                                      base_vec + (iota + c * LANES), mask=msk)
                off_s = off_s + plsc.all_reduce_population_count(msk)[0]
            return off_s

        ncand = lax.fori_loop(0, NVREG // 4, body_b, np.int32(0))
        nv = (ncand + LANES - 1) // LANES

        def count_pred(strict):
            def count(t):
                def cbody(j, acc):
                    u = ckey_ref[pl.ds(j * LANES, LANES)]
                    valid = (iota + j * LANES) < ncand
                    m = valid & ((u > t) if strict else (u >= t))
                    return acc + plsc.all_reduce_population_count(m)
                return jnp.max(lax.fori_loop(0, nv, cbody, zeros_i))
            return count

        count_ge = count_pred(False)
        count_gt = count_pred(True)

        def bbody(b, tu):
            cand_u = tu | (np.int32(1) << (np.int32(31) - b.astype(jnp.int32)))
            cand_key = cand_u ^ _I32_MIN
            ok = lax.cond(
                jnp.logical_or(cand_key > hi_key, cand_key < lo_key),
                lambda: cand_key < lo_key,
                lambda: count_ge(cand_key) >= TOPK)
            return jnp.where(ok, cand_u, tu)

        t_u = lax.fori_loop(0, 32, bbody, np.int32(0))
        thr = t_u ^ _I32_MIN
        tie_take = TOPK - count_gt(thr)

        def ebody(j, carry):
            eqoff, woff = carry
            u = ckey_ref[pl.ds(j * LANES, LANES)]
            iv = cidx_ref[pl.ds(j * LANES, LANES)]
            valid = (iota + j * LANES) < ncand
            gt = valid & (u > thr)
            eq = valid & (u == thr)
            eqrank = eqoff + plsc.cumsum(eq.astype(jnp.int32)) - 1
            sel = gt | (eq & (eqrank < tie_take))
            wpos = woff + plsc.cumsum(sel.astype(jnp.int32)) - 1
            plsc.store_scatter(wkey_ref, [wpos], u, mask=sel)
            plsc.store_scatter(widx_ref, [wpos], iv, mask=sel)
            return (eqoff + plsc.all_reduce_population_count(eq),
                    woff + plsc.all_reduce_population_count(sel))

        lax.fori_loop(0, nv, ebody, (zeros_i, zeros_i))

        kvs = [wkey_ref[pl.ds(w * LANES, LANES)] for w in range(4)]
        dvs = [widx_ref[pl.ds(w * LANES, LANES)] for w in range(4)]

        for w in range(4):
            val = jnp.maximum(_key_to_f32(kvs[w]), 0.0)
            plsc.store_scatter(row_ref, [dvs[w]], val)

        for i in range(TOPK):
            ks = kvs[i // LANES][i % LANES]
            dsc = dvs[i // LANES][i % LANES]
            cnt = zeros_i
            for w in range(4):
                gm = (kvs[w] > ks) | ((kvs[w] == ks) & (dvs[w] < dsc))
                cnt = cnt + plsc.all_reduce_population_count(gm)
            rank = jnp.max(cnt)
            plsc.store_scatter(oidx_ref, [zeros_i + rank], zeros_i + dsc,
                               mask=lane0)

        pltpu.sync_copy(row_ref, out_hbm.at[row])
        pltpu.sync_copy(oidx_ref, idx_hbm.at[row])
        return _carry

    lax.fori_loop(0, ROWS_PER_W, per_row, np.int32(0))


@functools.partial(jax.jit, donate_argnums=())
def _run(x):
    mesh = plsc.VectorSubcoreMesh(core_axis_name="c", subcore_axis_name="s")
    f = pl.kernel(
        _topk_body,
        out_type=[
            jax.ShapeDtypeStruct((ROWS, COLS), jnp.float32),
            jax.ShapeDtypeStruct((ROWS, TOPK), jnp.int32),
        ],
        mesh=mesh,
        compiler_params=pltpu.CompilerParams(needs_layout_passes=False),
        scratch_types=[
            pltpu.VMEM((COLS,), jnp.float32),
            pltpu.VMEM((COLS + LANES,), jnp.int32),
            pltpu.VMEM((COLS + LANES,), jnp.int32),
            pltpu.VMEM((TOPK,), jnp.int32),
            pltpu.VMEM((TOPK,), jnp.int32),
            pltpu.VMEM((TOPK,), jnp.int32),
        ],
    )
    res, idx = f(x)
    return res, idx


def kernel(x):
    return _run(x)
